# baseline (speedup 1.0000x reference)
import functools

import jax
import jax.numpy as jnp
from jax import lax
from jax.experimental import pallas as pl
from jax.experimental.pallas import tpu as pltpu
from jax.experimental.pallas import tpu_sc as plsc

M = 8
A = 1024
K = 64
CUT2 = 49.0
NBUCKET = 32
BUCKET_SCALE = NBUCKET / 160.0

NW = 32
ROWS_PER_W = (M * A) // NW
BLK = 8
NBLK = ROWS_PER_W // BLK


def _d2_tc(pos, posT, sqc, sqr):
    def body(p_ref, pt_ref, sc_ref, sr_ref, o_ref):
        dot = jnp.dot(p_ref[0], pt_ref[0], preferred_element_type=jnp.float32)
        d2 = (sc_ref[0] + sr_ref[0]) - 2.0 * dot
        d2 = jnp.maximum(d2, 0.0)
        r = lax.broadcasted_iota(jnp.int32, (A, A), 0)
        c = lax.broadcasted_iota(jnp.int32, (A, A), 1)
        o_ref[...] = d2 + jnp.where(r == c, 1e10, 0.0).astype(jnp.float32)

    return pl.pallas_call(
        body,
        grid=(M,),
        in_specs=[
            pl.BlockSpec((1, A, 3), lambda m: (m, 0, 0)),
            pl.BlockSpec((1, 3, A), lambda m: (m, 0, 0)),
            pl.BlockSpec((1, A, 1), lambda m: (m, 0, 0)),
            pl.BlockSpec((1, 1, A), lambda m: (m, 0, 0)),
        ],
        out_specs=pl.BlockSpec((A, A), lambda m: (m, 0)),
        out_shape=jax.ShapeDtypeStruct((M * A, A), jnp.float32),
    )(pos, posT, sqc, sqr)


def _splat_i32(x):
    return jnp.full((16,), 1, jnp.int32) * x


def _scalar(v16):
    return v16[0]


def _sc_select(d2f, px, py, pz):
    mesh = plsc.VectorSubcoreMesh(core_axis_name="c", subcore_axis_name="s",
                                  num_cores=2, num_subcores=16)

    @functools.partial(
        pl.kernel,
        out_type=[
            jax.ShapeDtypeStruct((M * A * K,), jnp.int32),
            jax.ShapeDtypeStruct((M * A * K,), jnp.float32),
            jax.ShapeDtypeStruct((M * A * K * 3,), jnp.float32),
            jax.ShapeDtypeStruct((M * A * K,), jnp.int32),
        ],
        mesh=mesh,
        scratch_types=[
            pltpu.VMEM((BLK, A), jnp.float32),
            pltpu.VMEM((BLK, A), jnp.float32),
            pltpu.VMEM((A,), jnp.float32),
            pltpu.VMEM((A,), jnp.float32),
            pltpu.VMEM((A,), jnp.float32),
            pltpu.VMEM((A + 16,), jnp.float32),
            pltpu.VMEM((A + 16,), jnp.int32),
            pltpu.VMEM((32 * NBUCKET,), jnp.int32),
            pltpu.VMEM((ROWS_PER_W * K,), jnp.int32),
            pltpu.VMEM((ROWS_PER_W * K,), jnp.float32),
            pltpu.VMEM((ROWS_PER_W * K * 3,), jnp.float32),
            pltpu.VMEM((ROWS_PER_W * K,), jnp.int32),
            pltpu.VMEM((K,), jnp.float32),
            pltpu.VMEM((K,), jnp.int32),
            pltpu.SemaphoreType.DMA,
            pltpu.SemaphoreType.DMA,
        ],
        compiler_params=pltpu.CompilerParams(needs_layout_passes=False),
    )
    def k(d2_hbm, px_hbm, py_hbm, pz_hbm,
          idx_hbm, dist_hbm, rij_hbm, msk_hbm,
          inbuf0, inbuf1, xbuf, ybuf, zbuf, keyb, idxb, cnt,
          idx_s, dist_s, rij_s, msk_s, fkb, fib, sem0, sem1):
        wid = lax.axis_index("s") * 2 + lax.axis_index("c")
        m = wid // 4
        r0 = wid * ROWS_PER_W
        moff = m * A

        iota = lax.iota(jnp.int32, 16)
        iota3 = iota * 3
        ones = jnp.full((16,), 1, jnp.int32)
        zeros16 = jnp.full((16,), 0, jnp.int32)
        inff = jnp.full((16,), 3.0e38, jnp.float32)

        pltpu.sync_copy(px_hbm.at[pl.ds(moff, A)], xbuf)
        pltpu.sync_copy(py_hbm.at[pl.ds(moff, A)], ybuf)
        pltpu.sync_copy(pz_hbm.at[pl.ds(moff, A)], zbuf)

        def row_body(rb, carry):
            blk_r0, inbuf = carry
            r = blk_r0 + rb
            a_loc = r - moff
            srow = r - r0

            def zero_body(i, _):
                cnt[pl.ds(i * 16, 16)] = zeros16
                return 0
            lax.fori_loop(0, 32 * NBUCKET // 16, zero_body, 0)

            laneoff = iota * NBUCKET
            def hist_body(c, _):
                for bank in range(2):
                    ch = c * 2 + bank
                    v = inbuf[rb, pl.ds(ch * 16, 16)]
                    b = jnp.minimum(v * BUCKET_SCALE,
                                    float(NBUCKET - 1)).astype(jnp.int32)
                    addr = b + laneoff + bank * (16 * NBUCKET)
                    old = plsc.load_gather(cnt, [addr])
                    plsc.store_scatter(cnt, [addr], old + ones)
                return 0
            lax.fori_loop(0, A // 32, hist_body, 0)

            def scan_body(i, carry2):
                tot, bfound = carry2
                def fold(l, s):
                    return s + cnt[pl.ds(l * NBUCKET + i * 16, 16)]
                ch = lax.fori_loop(0, 32, fold, zeros16)
                cum = plsc.cumsum(ch) + _splat_i32(tot)
                cross = cum >= K
                anyc = _scalar(cross.astype(jnp.int32))
                ffs = _scalar(plsc.all_reduce_ffs(cross))
                cand = i * 16 + ffs
                bnew = jnp.where((anyc > 0) & (bfound >= NBUCKET),
                                 cand, bfound)
                return cum[15], bnew
            _, bsel = lax.fori_loop(0, NBUCKET // 16, scan_body,
                                    (jnp.int32(0), jnp.int32(NBUCKET)))
            bsel_v = _splat_i32(bsel)

            def comp_body(c, off):
                v = inbuf[rb, pl.ds(c * 16, 16)]
                b = jnp.minimum(v * BUCKET_SCALE,
                                float(NBUCKET - 1)).astype(jnp.int32)
                mk = b <= bsel_v
                plsc.store_compressed(keyb.at[pl.ds(off, 16)], v, mask=mk)
                iv = iota + _splat_i32(c * 16)
                plsc.store_compressed(idxb.at[pl.ds(off, 16)], iv, mask=mk)
                pc = _scalar(plsc.all_reduce_population_count(mk))
                return off + pc
            csz = lax.fori_loop(0, A // 16, comp_body, jnp.int32(0))

            c0 = csz // 16
            rem = csz - c0 * 16
            remv = _splat_i32(rem)
            tk = keyb[pl.ds(c0 * 16, 16)]
            keyb[pl.ds(c0 * 16, 16)] = jnp.where(iota >= remv, inff, tk)
            ti = idxb[pl.ds(c0 * 16, 16)]
            idxb[pl.ds(c0 * 16, 16)] = jnp.where(iota >= remv, zeros16, ti)
            nch = (csz + 15) // 16

            def merge_body(c, acc):
                ak0, ai0, ak1, ai1, ak2, ai2, ak3, ai3 = acc
                ck, ci = plsc.sort_key_val(keyb[pl.ds(c * 16, 16)],
                                           idxb[pl.ds(c * 16, 16)])
                outs = []
                for (akj, aij) in ((ak0, ai0), (ak1, ai1),
                                   (ak2, ai2), (ak3, ai3)):
                    rk = lax.rev(ck, (0,))
                    ri = lax.rev(ci, (0,))
                    a_le = (akj < rk) | ((akj == rk) & (aij <= ri))
                    lo_k = jnp.where(a_le, akj, rk)
                    lo_i = jnp.where(a_le, aij, ri)
                    hi_k = jnp.where(a_le, rk, akj)
                    hi_i = jnp.where(a_le, ri, aij)
                    nk, ni = plsc.sort_key_val(lo_k, lo_i)
                    ck, ci = plsc.sort_key_val(hi_k, hi_i)
                    outs.append((nk, ni))
                return (outs[0][0], outs[0][1], outs[1][0], outs[1][1],
                        outs[2][0], outs[2][1], outs[3][0], outs[3][1])
            init = (inff, zeros16, inff, zeros16,
                    inff, zeros16, inff, zeros16)
            acc = lax.fori_loop(0, nch, merge_body, init)

            aks = (acc[0], acc[2], acc[4], acc[6])
            ais = (acc[1], acc[3], acc[5], acc[7])
            for t in range(4):
                fkb[pl.ds(t * 16, 16)] = aks[t]
                fib[pl.ds(t * 16, 16)] = ais[t]
            carry = jnp.int32(-1)
            key2 = []
            nties = jnp.int32(0)
            for t in range(4):
                pidx = jnp.maximum(iota + (16 * t - 1), 0)
                prev = plsc.load_gather(fkb, [pidx])
                newrun = aks[t] != prev
                if t == 0:
                    newrun = newrun | (iota < 1)
                nties = nties + _scalar(
                    plsc.all_reduce_population_count(~newrun))
                gp = iota + _splat_i32(16 * t)
                s = jnp.where(newrun, gp, -1)
                r0v = jnp.maximum(plsc.cummax(s), _splat_i32(carry))
                carry = _scalar(r0v)
                key2.append(r0v * 2048 + ais[t])

            def fixup(ops):
                key2a, key2b, key2c, key2d = ops
                big = _splat_i32(1 << 30)
                b0, b1, b2, b3 = big, big, big, big
                p0 = p1 = p2 = p3 = zeros16
                for t, k2 in enumerate((key2a, key2b, key2c, key2d)):
                    gp = iota + _splat_i32(16 * t)
                    ck2, cp2 = plsc.sort_key_val(k2, gp)
                    for j in range(4):
                        bj = (b0, b1, b2, b3)[j]
                        pj = (p0, p1, p2, p3)[j]
                        rk2 = lax.rev(ck2, (0,))
                        rp2 = lax.rev(cp2, (0,))
                        a_le = bj <= rk2
                        lo_k = jnp.where(a_le, bj, rk2)
                        lo_p = jnp.where(a_le, pj, rp2)
                        hi_k = jnp.where(a_le, rk2, bj)
                        hi_p = jnp.where(a_le, rp2, pj)
                        nk2, np2 = plsc.sort_key_val(lo_k, lo_p)
                        ck2, cp2 = plsc.sort_key_val(hi_k, hi_p)
                        if j == 0:
                            b0, p0 = nk2, np2
                        elif j == 1:
                            b1, p1 = nk2, np2
                        elif j == 2:
                            b2, p2 = nk2, np2
                        else:
                            b3, p3 = nk2, np2
                    del ck2, cp2
                fin = []
                for pj in (p0, p1, p2, p3):
                    kf = plsc.load_gather(fkb, [pj])
                    vf = plsc.load_gather(fib, [pj])
                    fin.extend((kf, vf))
                return tuple(fin)

            acc = lax.cond(nties > 0, fixup, lambda ops: acc, tuple(key2))

            av = _splat_i32(a_loc)
            xa = plsc.load_gather(xbuf, [av])
            ya = plsc.load_gather(ybuf, [av])
            za = plsc.load_gather(zbuf, [av])
            obase = srow * K
            rbase = srow * K * 3
            for t in range(4):
                kt = acc[2 * t]
                it = acc[2 * t + 1]
                idx_s[pl.ds(obase + t * 16, 16)] = it + _splat_i32(moff)
                mk = kt < CUT2
                msk_s[pl.ds(obase + t * 16, 16)] = mk.astype(jnp.int32)
                kc = jnp.maximum(kt, 1e-30)
                u = plsc.bitcast(kc, jnp.int32)
                y = plsc.bitcast(_splat_i32(0x5F3759DF) -
                                 lax.shift_right_logical(u, 1), jnp.float32)
                half = kc * (-0.5)
                for _ in range(3):
                    y = y * (half * y * y + 1.5)
                d = kt * y
                dist_s[pl.ds(obase + t * 16, 16)] = jnp.where(mk, d, 0.0)
                mf = jnp.where(mk, 1.0, 0.0)
                sb = _splat_i32(rbase + t * 48) + iota3
                xg = plsc.load_gather(xbuf, [it])
                plsc.store_scatter(rij_s, [sb], (xg - xa) * mf)
                yg = plsc.load_gather(ybuf, [it])
                plsc.store_scatter(rij_s, [sb + ones], (yg - ya) * mf)
                zg = plsc.load_gather(zbuf, [it])
                plsc.store_scatter(rij_s, [sb + ones + ones], (zg - za) * mf)
            return carry

        maxr0 = M * A - BLK
        cp0 = pltpu.make_async_copy(d2_hbm.at[pl.ds(r0, BLK)], inbuf0, sem0)
        cp0.start()

        def blk2_body(h, _):
            b0r = r0 + (2 * h) * BLK
            b1r = r0 + (2 * h + 1) * BLK
            b2r = jnp.minimum(r0 + (2 * h + 2) * BLK, maxr0)
            pltpu.make_async_copy(
                d2_hbm.at[pl.ds(b0r, BLK)], inbuf0, sem0).wait()
            pltpu.make_async_copy(
                d2_hbm.at[pl.ds(b1r, BLK)], inbuf1, sem1).start()
            def rb0(rb, c):
                row_body(rb, (c[0], inbuf0))
                return c
            lax.fori_loop(0, BLK, rb0, (b0r,))
            pltpu.make_async_copy(
                d2_hbm.at[pl.ds(b1r, BLK)], inbuf1, sem1).wait()
            pltpu.make_async_copy(
                d2_hbm.at[pl.ds(b2r, BLK)], inbuf0, sem0).start()
            def rb1(rb, c):
                row_body(rb, (c[0], inbuf1))
                return c
            lax.fori_loop(0, BLK, rb1, (b1r,))
            return 0
        lax.fori_loop(0, NBLK // 2, blk2_body, 0)
        pltpu.make_async_copy(d2_hbm.at[pl.ds(maxr0, BLK)], inbuf0, sem0).wait()

        pltpu.sync_copy(idx_s, idx_hbm.at[pl.ds(r0 * K, ROWS_PER_W * K)])
        pltpu.sync_copy(dist_s, dist_hbm.at[pl.ds(r0 * K, ROWS_PER_W * K)])
        pltpu.sync_copy(rij_s, rij_hbm.at[pl.ds(r0 * K * 3, ROWS_PER_W * K * 3)])
        pltpu.sync_copy(msk_s, msk_hbm.at[pl.ds(r0 * K, ROWS_PER_W * K)])

    return k(d2f, px, py, pz)


def kernel(atom_types, positions, n_atoms, cells, pbc, n_molecules):
    pos = positions.reshape(M, A, 3)
    sq = jnp.sum(pos * pos, axis=-1)
    d2 = _d2_tc(pos, jnp.swapaxes(pos, 1, 2), sq[..., None], sq[:, None, :])

    px = pos[:, :, 0].reshape(-1)
    py = pos[:, :, 1].reshape(-1)
    pz = pos[:, :, 2].reshape(-1)
    idx_j, dist, rij, msk = _sc_select(d2, px, py, pz)

    idx_i = jnp.repeat(jnp.arange(M * A, dtype=jnp.int32), K)
    return (idx_i,
            idx_j,
            rij.reshape(M, A, K, 3),
            dist.reshape(M, A, K),
            msk.reshape(M, A, K).astype(bool))


# speedup vs baseline: 1.0254x; 1.0254x over previous
import functools

import jax
import jax.numpy as jnp
from jax import lax
from jax.experimental import pallas as pl
from jax.experimental.pallas import tpu as pltpu
from jax.experimental.pallas import tpu_sc as plsc

M = 8
A = 1024
K = 64
CUT2 = 49.0
NBUCKET = 32
BUCKET_SCALE = NBUCKET / 160.0

NW = 32
ROWS_PER_W = (M * A) // NW
BLK = 8
NBLK = ROWS_PER_W // BLK


def _d2_tc(pos, posT, sqc, sqr):
    def body(p_ref, pt_ref, sc_ref, sr_ref, o_ref):
        dot = jnp.dot(p_ref[0], pt_ref[0], preferred_element_type=jnp.float32)
        d2 = (sc_ref[0] + sr_ref[0]) - 2.0 * dot
        d2 = jnp.maximum(d2, 0.0)
        r = lax.broadcasted_iota(jnp.int32, (A, A), 0)
        c = lax.broadcasted_iota(jnp.int32, (A, A), 1)
        o_ref[...] = d2 + jnp.where(r == c, 1e10, 0.0).astype(jnp.float32)

    return pl.pallas_call(
        body,
        grid=(M,),
        in_specs=[
            pl.BlockSpec((1, A, 3), lambda m: (m, 0, 0)),
            pl.BlockSpec((1, 3, A), lambda m: (m, 0, 0)),
            pl.BlockSpec((1, A, 1), lambda m: (m, 0, 0)),
            pl.BlockSpec((1, 1, A), lambda m: (m, 0, 0)),
        ],
        out_specs=pl.BlockSpec((A, A), lambda m: (m, 0)),
        out_shape=jax.ShapeDtypeStruct((M * A, A), jnp.float32),
    )(pos, posT, sqc, sqr)


def _splat_i32(x):
    return jnp.full((16,), 1, jnp.int32) * x


def _scalar(v16):
    return v16[0]


def _sc_select(d2f, px, py, pz):
    mesh = plsc.VectorSubcoreMesh(core_axis_name="c", subcore_axis_name="s",
                                  num_cores=2, num_subcores=16)

    @functools.partial(
        pl.kernel,
        out_type=[
            jax.ShapeDtypeStruct((M * A * K,), jnp.int32),
            jax.ShapeDtypeStruct((M * A * K,), jnp.float32),
            jax.ShapeDtypeStruct((M * A * K * 3,), jnp.float32),
            jax.ShapeDtypeStruct((M * A * K,), jnp.int32),
        ],
        mesh=mesh,
        scratch_types=[
            pltpu.VMEM((BLK, A), jnp.float32),
            pltpu.VMEM((BLK, A), jnp.float32),
            pltpu.VMEM((A,), jnp.float32),
            pltpu.VMEM((A,), jnp.float32),
            pltpu.VMEM((A,), jnp.float32),
            pltpu.VMEM((A + 16,), jnp.float32),
            pltpu.VMEM((A + 16,), jnp.int32),
            pltpu.VMEM((32 * NBUCKET,), jnp.int32),
            pltpu.VMEM((ROWS_PER_W * K,), jnp.int32),
            pltpu.VMEM((ROWS_PER_W * K,), jnp.float32),
            pltpu.VMEM((ROWS_PER_W * K * 3,), jnp.float32),
            pltpu.VMEM((ROWS_PER_W * K,), jnp.int32),
            pltpu.VMEM((K,), jnp.float32),
            pltpu.VMEM((K,), jnp.int32),
            pltpu.SemaphoreType.DMA,
            pltpu.SemaphoreType.DMA,
        ],
        compiler_params=pltpu.CompilerParams(needs_layout_passes=False),
    )
    def k(d2_hbm, px_hbm, py_hbm, pz_hbm,
          idx_hbm, dist_hbm, rij_hbm, msk_hbm,
          inbuf0, inbuf1, xbuf, ybuf, zbuf, keyb, idxb, cnt,
          idx_s, dist_s, rij_s, msk_s, fkb, fib, sem0, sem1):
        wid = lax.axis_index("s") * 2 + lax.axis_index("c")
        m = wid // 4
        r0 = wid * ROWS_PER_W
        moff = m * A

        iota = lax.iota(jnp.int32, 16)
        iota3 = iota * 3
        ones = jnp.full((16,), 1, jnp.int32)
        zeros16 = jnp.full((16,), 0, jnp.int32)
        inff = jnp.full((16,), 3.0e38, jnp.float32)

        pltpu.sync_copy(px_hbm.at[pl.ds(moff, A)], xbuf)
        pltpu.sync_copy(py_hbm.at[pl.ds(moff, A)], ybuf)
        pltpu.sync_copy(pz_hbm.at[pl.ds(moff, A)], zbuf)

        def row_body(rb, carry):
            blk_r0, inbuf = carry
            r = blk_r0 + rb
            a_loc = r - moff
            srow = r - r0

            def zero_body(i, _):
                cnt[pl.ds(i * 16, 16)] = zeros16
                return 0
            lax.fori_loop(0, 32 * NBUCKET // 16, zero_body, 0)

            laneoff = iota * NBUCKET
            def hist_body(c, _):
                for bank in range(2):
                    ch = c * 2 + bank
                    v = inbuf[rb, pl.ds(ch * 16, 16)]
                    b = jnp.minimum(v * BUCKET_SCALE,
                                    float(NBUCKET - 1)).astype(jnp.int32)
                    addr = b + laneoff + bank * (16 * NBUCKET)
                    old = plsc.load_gather(cnt, [addr])
                    plsc.store_scatter(cnt, [addr], old + ones)
                return 0
            lax.fori_loop(0, A // 32, hist_body, 0)

            def scan_body(i, carry2):
                tot, bfound = carry2
                def fold(l, s):
                    return s + cnt[pl.ds(l * NBUCKET + i * 16, 16)]
                ch = lax.fori_loop(0, 32, fold, zeros16)
                cum = plsc.cumsum(ch) + _splat_i32(tot)
                cross = cum >= K
                anyc = cum[15] >= K
                ffs = _scalar(plsc.all_reduce_ffs(cross))
                cand = i * 16 + ffs
                bnew = jnp.where(anyc & (bfound >= NBUCKET),
                                 cand, bfound)
                return cum[15], bnew
            _, bsel = lax.fori_loop(0, NBUCKET // 16, scan_body,
                                    (jnp.int32(0), jnp.int32(NBUCKET)))
            bsel_v = _splat_i32(bsel)

            def comp_body(c, off):
                v = inbuf[rb, pl.ds(c * 16, 16)]
                b = jnp.minimum(v * BUCKET_SCALE,
                                float(NBUCKET - 1)).astype(jnp.int32)
                mk = b <= bsel_v
                plsc.store_compressed(keyb.at[pl.ds(off, 16)], v, mask=mk)
                iv = iota + _splat_i32(c * 16)
                plsc.store_compressed(idxb.at[pl.ds(off, 16)], iv, mask=mk)
                pc = _scalar(plsc.all_reduce_population_count(mk))
                return off + pc
            csz = lax.fori_loop(0, A // 16, comp_body, jnp.int32(0))

            c0 = csz // 16
            rem = csz - c0 * 16
            remv = _splat_i32(rem)
            tk = keyb[pl.ds(c0 * 16, 16)]
            keyb[pl.ds(c0 * 16, 16)] = jnp.where(iota >= remv, inff, tk)
            ti = idxb[pl.ds(c0 * 16, 16)]
            idxb[pl.ds(c0 * 16, 16)] = jnp.where(iota >= remv, zeros16, ti)
            nch = (csz + 15) // 16

            def merge_body(c, acc):
                ak0, ai0, ak1, ai1, ak2, ai2, ak3, ai3 = acc
                ck, ci = plsc.sort_key_val(keyb[pl.ds(c * 16, 16)],
                                           idxb[pl.ds(c * 16, 16)])
                outs = []
                for (akj, aij) in ((ak0, ai0), (ak1, ai1),
                                   (ak2, ai2), (ak3, ai3)):
                    rk = lax.rev(ck, (0,))
                    ri = lax.rev(ci, (0,))
                    a_le = (akj < rk) | ((akj == rk) & (aij <= ri))
                    lo_k = jnp.where(a_le, akj, rk)
                    lo_i = jnp.where(a_le, aij, ri)
                    hi_k = jnp.where(a_le, rk, akj)
                    hi_i = jnp.where(a_le, ri, aij)
                    nk, ni = plsc.sort_key_val(lo_k, lo_i)
                    ck, ci = plsc.sort_key_val(hi_k, hi_i)
                    outs.append((nk, ni))
                return (outs[0][0], outs[0][1], outs[1][0], outs[1][1],
                        outs[2][0], outs[2][1], outs[3][0], outs[3][1])
            init = (inff, zeros16, inff, zeros16,
                    inff, zeros16, inff, zeros16)
            acc = lax.fori_loop(0, nch, merge_body, init)

            aks = (acc[0], acc[2], acc[4], acc[6])
            ais = (acc[1], acc[3], acc[5], acc[7])
            for t in range(4):
                fkb[pl.ds(t * 16, 16)] = aks[t]
                fib[pl.ds(t * 16, 16)] = ais[t]
            carry = jnp.int32(-1)
            key2 = []
            nties = jnp.int32(0)
            for t in range(4):
                pidx = jnp.maximum(iota + (16 * t - 1), 0)
                prev = plsc.load_gather(fkb, [pidx])
                newrun = aks[t] != prev
                if t == 0:
                    newrun = newrun | (iota < 1)
                nties = nties + _scalar(
                    plsc.all_reduce_population_count(~newrun))
                gp = iota + _splat_i32(16 * t)
                s = jnp.where(newrun, gp, -1)
                r0v = jnp.maximum(plsc.cummax(s), _splat_i32(carry))
                carry = r0v[15]
                key2.append(r0v * 2048 + ais[t])

            def fixup(ops):
                key2a, key2b, key2c, key2d = ops
                big = _splat_i32(1 << 30)
                b0, b1, b2, b3 = big, big, big, big
                p0 = p1 = p2 = p3 = zeros16
                for t, k2 in enumerate((key2a, key2b, key2c, key2d)):
                    gp = iota + _splat_i32(16 * t)
                    ck2, cp2 = plsc.sort_key_val(k2, gp)
                    for j in range(4):
                        bj = (b0, b1, b2, b3)[j]
                        pj = (p0, p1, p2, p3)[j]
                        rk2 = lax.rev(ck2, (0,))
                        rp2 = lax.rev(cp2, (0,))
                        a_le = bj <= rk2
                        lo_k = jnp.where(a_le, bj, rk2)
                        lo_p = jnp.where(a_le, pj, rp2)
                        hi_k = jnp.where(a_le, rk2, bj)
                        hi_p = jnp.where(a_le, rp2, pj)
                        nk2, np2 = plsc.sort_key_val(lo_k, lo_p)
                        ck2, cp2 = plsc.sort_key_val(hi_k, hi_p)
                        if j == 0:
                            b0, p0 = nk2, np2
                        elif j == 1:
                            b1, p1 = nk2, np2
                        elif j == 2:
                            b2, p2 = nk2, np2
                        else:
                            b3, p3 = nk2, np2
                    del ck2, cp2
                fin = []
                for pj in (p0, p1, p2, p3):
                    kf = plsc.load_gather(fkb, [pj])
                    vf = plsc.load_gather(fib, [pj])
                    fin.extend((kf, vf))
                return tuple(fin)

            acc = lax.cond(nties > 0, fixup, lambda ops: acc, tuple(key2))

            av = _splat_i32(a_loc)
            xa = plsc.load_gather(xbuf, [av])
            ya = plsc.load_gather(ybuf, [av])
            za = plsc.load_gather(zbuf, [av])
            obase = srow * K
            rbase = srow * K * 3
            for t in range(4):
                kt = acc[2 * t]
                it = acc[2 * t + 1]
                idx_s[pl.ds(obase + t * 16, 16)] = it + _splat_i32(moff)
                mk = kt < CUT2
                msk_s[pl.ds(obase + t * 16, 16)] = mk.astype(jnp.int32)
                kc = jnp.maximum(kt, 1e-30)
                u = plsc.bitcast(kc, jnp.int32)
                y = plsc.bitcast(_splat_i32(0x5F3759DF) -
                                 lax.shift_right_logical(u, 1), jnp.float32)
                half = kc * (-0.5)
                for _ in range(3):
                    y = y * (half * y * y + 1.5)
                d = kt * y
                dist_s[pl.ds(obase + t * 16, 16)] = jnp.where(mk, d, 0.0)
                mf = jnp.where(mk, 1.0, 0.0)
                sb = _splat_i32(rbase + t * 48) + iota3
                xg = plsc.load_gather(xbuf, [it])
                plsc.store_scatter(rij_s, [sb], (xg - xa) * mf)
                yg = plsc.load_gather(ybuf, [it])
                plsc.store_scatter(rij_s, [sb + ones], (yg - ya) * mf)
                zg = plsc.load_gather(zbuf, [it])
                plsc.store_scatter(rij_s, [sb + ones + ones], (zg - za) * mf)
            return carry

        maxr0 = M * A - BLK
        cp0 = pltpu.make_async_copy(d2_hbm.at[pl.ds(r0, BLK)], inbuf0, sem0)
        cp0.start()

        def blk2_body(h, _):
            b0r = r0 + (2 * h) * BLK
            b1r = r0 + (2 * h + 1) * BLK
            b2r = jnp.minimum(r0 + (2 * h + 2) * BLK, maxr0)
            pltpu.make_async_copy(
                d2_hbm.at[pl.ds(b0r, BLK)], inbuf0, sem0).wait()
            pltpu.make_async_copy(
                d2_hbm.at[pl.ds(b1r, BLK)], inbuf1, sem1).start()
            def rb0(rb, c):
                row_body(rb, (c[0], inbuf0))
                return c
            lax.fori_loop(0, BLK, rb0, (b0r,))
            pltpu.make_async_copy(
                d2_hbm.at[pl.ds(b1r, BLK)], inbuf1, sem1).wait()
            pltpu.make_async_copy(
                d2_hbm.at[pl.ds(b2r, BLK)], inbuf0, sem0).start()
            def rb1(rb, c):
                row_body(rb, (c[0], inbuf1))
                return c
            lax.fori_loop(0, BLK, rb1, (b1r,))
            return 0
        lax.fori_loop(0, NBLK // 2, blk2_body, 0)
        pltpu.make_async_copy(d2_hbm.at[pl.ds(maxr0, BLK)], inbuf0, sem0).wait()

        pltpu.sync_copy(idx_s, idx_hbm.at[pl.ds(r0 * K, ROWS_PER_W * K)])
        pltpu.sync_copy(dist_s, dist_hbm.at[pl.ds(r0 * K, ROWS_PER_W * K)])
        pltpu.sync_copy(rij_s, rij_hbm.at[pl.ds(r0 * K * 3, ROWS_PER_W * K * 3)])
        pltpu.sync_copy(msk_s, msk_hbm.at[pl.ds(r0 * K, ROWS_PER_W * K)])

    return k(d2f, px, py, pz)


def kernel(atom_types, positions, n_atoms, cells, pbc, n_molecules):
    pos = positions.reshape(M, A, 3)
    sq = jnp.sum(pos * pos, axis=-1)
    d2 = _d2_tc(pos, jnp.swapaxes(pos, 1, 2), sq[..., None], sq[:, None, :])

    px = pos[:, :, 0].reshape(-1)
    py = pos[:, :, 1].reshape(-1)
    pz = pos[:, :, 2].reshape(-1)
    idx_j, dist, rij, msk = _sc_select(d2, px, py, pz)

    idx_i = jnp.repeat(jnp.arange(M * A, dtype=jnp.int32), K)
    return (idx_i,
            idx_j,
            rij.reshape(M, A, K, 3),
            dist.reshape(M, A, K),
            msk.reshape(M, A, K).astype(bool))


# speedup vs baseline: 1.1130x; 1.0854x over previous
import functools

import jax
import jax.numpy as jnp
from jax import lax
from jax.experimental import pallas as pl
from jax.experimental.pallas import tpu as pltpu
from jax.experimental.pallas import tpu_sc as plsc

M = 8
A = 1024
K = 64
CUT2 = 49.0
NBUCKET = 32
BUCKET_SCALE = NBUCKET / 160.0

NW = 32
ROWS_PER_W = (M * A) // NW
BLK = 8
NBLK = ROWS_PER_W // BLK


def _d2_tc(pos, posT, sqc, sqr):
    def body(p_ref, pt_ref, sc_ref, sr_ref, o_ref):
        dot = jnp.dot(p_ref[0], pt_ref[0], preferred_element_type=jnp.float32)
        d2 = (sc_ref[0] + sr_ref[0]) - 2.0 * dot
        d2 = jnp.maximum(d2, 0.0)
        r = lax.broadcasted_iota(jnp.int32, (A, A), 0)
        c = lax.broadcasted_iota(jnp.int32, (A, A), 1)
        o_ref[...] = d2 + jnp.where(r == c, 1e10, 0.0).astype(jnp.float32)

    return pl.pallas_call(
        body,
        grid=(M,),
        in_specs=[
            pl.BlockSpec((1, A, 3), lambda m: (m, 0, 0)),
            pl.BlockSpec((1, 3, A), lambda m: (m, 0, 0)),
            pl.BlockSpec((1, A, 1), lambda m: (m, 0, 0)),
            pl.BlockSpec((1, 1, A), lambda m: (m, 0, 0)),
        ],
        out_specs=pl.BlockSpec((A, A), lambda m: (m, 0)),
        out_shape=jax.ShapeDtypeStruct((M * A, A), jnp.float32),
    )(pos, posT, sqc, sqr)


def _splat_i32(x):
    return jnp.full((16,), 1, jnp.int32) * x


def _scalar(v16):
    return v16[0]


def _sc_select(d2f, px, py, pz):
    mesh = plsc.VectorSubcoreMesh(core_axis_name="c", subcore_axis_name="s",
                                  num_cores=2, num_subcores=16)

    @functools.partial(
        pl.kernel,
        out_type=[
            jax.ShapeDtypeStruct((M * A * K,), jnp.int32),
            jax.ShapeDtypeStruct((M * A * K,), jnp.float32),
            jax.ShapeDtypeStruct((M * A * K * 3,), jnp.float32),
            jax.ShapeDtypeStruct((M * A * K,), jnp.int32),
        ],
        mesh=mesh,
        scratch_types=[
            pltpu.VMEM((BLK, A), jnp.float32),
            pltpu.VMEM((BLK, A), jnp.float32),
            pltpu.VMEM((A,), jnp.float32),
            pltpu.VMEM((A,), jnp.float32),
            pltpu.VMEM((A,), jnp.float32),
            pltpu.VMEM((A + 16,), jnp.float32),
            pltpu.VMEM((A + 16,), jnp.int32),
            pltpu.VMEM((16 * NBUCKET,), jnp.int32),
            pltpu.VMEM((ROWS_PER_W * K,), jnp.int32),
            pltpu.VMEM((ROWS_PER_W * K,), jnp.float32),
            pltpu.VMEM((ROWS_PER_W * K * 3,), jnp.float32),
            pltpu.VMEM((ROWS_PER_W * K,), jnp.int32),
            pltpu.VMEM((K,), jnp.float32),
            pltpu.VMEM((K,), jnp.int32),
            pltpu.SemaphoreType.DMA,
            pltpu.SemaphoreType.DMA,
        ],
        compiler_params=pltpu.CompilerParams(needs_layout_passes=False),
    )
    def k(d2_hbm, px_hbm, py_hbm, pz_hbm,
          idx_hbm, dist_hbm, rij_hbm, msk_hbm,
          inbuf0, inbuf1, xbuf, ybuf, zbuf, keyb, idxb, cnt,
          idx_s, dist_s, rij_s, msk_s, fkb, fib, sem0, sem1):
        wid = lax.axis_index("s") * 2 + lax.axis_index("c")
        m = wid // 4
        r0 = wid * ROWS_PER_W
        moff = m * A

        iota = lax.iota(jnp.int32, 16)
        iota3 = iota * 3
        ones = jnp.full((16,), 1, jnp.int32)
        zeros16 = jnp.full((16,), 0, jnp.int32)
        inff = jnp.full((16,), 3.0e38, jnp.float32)

        pltpu.sync_copy(px_hbm.at[pl.ds(moff, A)], xbuf)
        pltpu.sync_copy(py_hbm.at[pl.ds(moff, A)], ybuf)
        pltpu.sync_copy(pz_hbm.at[pl.ds(moff, A)], zbuf)

        def row_body(rb, carry):
            blk_r0, inbuf = carry
            r = blk_r0 + rb
            a_loc = r - moff
            srow = r - r0

            def zero_body(i, _):
                cnt[pl.ds(i * 16, 16)] = zeros16
                return 0
            lax.fori_loop(0, 16 * NBUCKET // 16, zero_body, 0)

            laneoff = iota * NBUCKET
            def hist_body(c, _):
                v = inbuf[rb, pl.ds(c * 16, 16)]
                b = jnp.minimum(v * BUCKET_SCALE,
                                float(NBUCKET - 1)).astype(jnp.int32)
                addr = b + laneoff
                old = plsc.load_gather(cnt, [addr])
                plsc.store_scatter(cnt, [addr], old + ones)
                return 0
            lax.fori_loop(0, A // 16, hist_body, 0)

            def scan_body(i, carry2):
                tot, bfound = carry2
                def fold(l, s):
                    return s + cnt[pl.ds(l * NBUCKET + i * 16, 16)]
                ch = lax.fori_loop(0, 16, fold, zeros16)
                cum = plsc.cumsum(ch) + _splat_i32(tot)
                cross = cum >= K
                anyc = cum[15] >= K
                ffs = _scalar(plsc.all_reduce_ffs(cross))
                cand = i * 16 + ffs
                bnew = jnp.where(anyc & (bfound >= NBUCKET),
                                 cand, bfound)
                return cum[15], bnew
            _, bsel = lax.fori_loop(0, NBUCKET // 16, scan_body,
                                    (jnp.int32(0), jnp.int32(NBUCKET)))
            bsel_v = _splat_i32(bsel)

            def comp_body(c, off):
                v = inbuf[rb, pl.ds(c * 16, 16)]
                b = jnp.minimum(v * BUCKET_SCALE,
                                float(NBUCKET - 1)).astype(jnp.int32)
                mk = b <= bsel_v
                plsc.store_compressed(keyb.at[pl.ds(off, 16)], v, mask=mk)
                iv = iota + _splat_i32(c * 16)
                plsc.store_compressed(idxb.at[pl.ds(off, 16)], iv, mask=mk)
                pc = _scalar(plsc.all_reduce_population_count(mk))
                return off + pc
            csz = lax.fori_loop(0, A // 16, comp_body, jnp.int32(0))

            c0 = csz // 16
            rem = csz - c0 * 16
            remv = _splat_i32(rem)
            tk = keyb[pl.ds(c0 * 16, 16)]
            keyb[pl.ds(c0 * 16, 16)] = jnp.where(iota >= remv, inff, tk)
            ti = idxb[pl.ds(c0 * 16, 16)]
            idxb[pl.ds(c0 * 16, 16)] = jnp.where(iota >= remv, zeros16, ti)
            nch = (csz + 15) // 16

            def merge_body(c, acc):
                ak0, ai0, ak1, ai1, ak2, ai2, ak3, ai3 = acc
                ck, ci = plsc.sort_key_val(keyb[pl.ds(c * 16, 16)],
                                           idxb[pl.ds(c * 16, 16)])
                outs = []
                for (akj, aij) in ((ak0, ai0), (ak1, ai1),
                                   (ak2, ai2), (ak3, ai3)):
                    rk = lax.rev(ck, (0,))
                    ri = lax.rev(ci, (0,))
                    a_le = (akj < rk) | ((akj == rk) & (aij <= ri))
                    lo_k = jnp.where(a_le, akj, rk)
                    lo_i = jnp.where(a_le, aij, ri)
                    hi_k = jnp.where(a_le, rk, akj)
                    hi_i = jnp.where(a_le, ri, aij)
                    nk, ni = plsc.sort_key_val(lo_k, lo_i)
                    ck, ci = plsc.sort_key_val(hi_k, hi_i)
                    outs.append((nk, ni))
                return (outs[0][0], outs[0][1], outs[1][0], outs[1][1],
                        outs[2][0], outs[2][1], outs[3][0], outs[3][1])
            init = (inff, zeros16, inff, zeros16,
                    inff, zeros16, inff, zeros16)
            acc = lax.fori_loop(0, nch, merge_body, init)

            aks = (acc[0], acc[2], acc[4], acc[6])
            ais = (acc[1], acc[3], acc[5], acc[7])
            for t in range(4):
                fkb[pl.ds(t * 16, 16)] = aks[t]
                fib[pl.ds(t * 16, 16)] = ais[t]
            carry = jnp.int32(-1)
            key2 = []
            nties = jnp.int32(0)
            for t in range(4):
                pidx = jnp.maximum(iota + (16 * t - 1), 0)
                prev = plsc.load_gather(fkb, [pidx])
                newrun = aks[t] != prev
                if t == 0:
                    newrun = newrun | (iota < 1)
                nties = nties + _scalar(
                    plsc.all_reduce_population_count(~newrun))
                gp = iota + _splat_i32(16 * t)
                s = jnp.where(newrun, gp, -1)
                r0v = jnp.maximum(plsc.cummax(s), _splat_i32(carry))
                carry = r0v[15]
                key2.append(r0v * 2048 + ais[t])

            def fixup(ops):
                key2a, key2b, key2c, key2d = ops
                big = _splat_i32(1 << 30)
                b0, b1, b2, b3 = big, big, big, big
                p0 = p1 = p2 = p3 = zeros16
                for t, k2 in enumerate((key2a, key2b, key2c, key2d)):
                    gp = iota + _splat_i32(16 * t)
                    ck2, cp2 = plsc.sort_key_val(k2, gp)
                    for j in range(4):
                        bj = (b0, b1, b2, b3)[j]
                        pj = (p0, p1, p2, p3)[j]
                        rk2 = lax.rev(ck2, (0,))
                        rp2 = lax.rev(cp2, (0,))
                        a_le = bj <= rk2
                        lo_k = jnp.where(a_le, bj, rk2)
                        lo_p = jnp.where(a_le, pj, rp2)
                        hi_k = jnp.where(a_le, rk2, bj)
                        hi_p = jnp.where(a_le, rp2, pj)
                        nk2, np2 = plsc.sort_key_val(lo_k, lo_p)
                        ck2, cp2 = plsc.sort_key_val(hi_k, hi_p)
                        if j == 0:
                            b0, p0 = nk2, np2
                        elif j == 1:
                            b1, p1 = nk2, np2
                        elif j == 2:
                            b2, p2 = nk2, np2
                        else:
                            b3, p3 = nk2, np2
                    del ck2, cp2
                fin = []
                for pj in (p0, p1, p2, p3):
                    kf = plsc.load_gather(fkb, [pj])
                    vf = plsc.load_gather(fib, [pj])
                    fin.extend((kf, vf))
                return tuple(fin)

            acc = lax.cond(nties > 0, fixup, lambda ops: acc, tuple(key2))

            av = _splat_i32(a_loc)
            xa = plsc.load_gather(xbuf, [av])
            ya = plsc.load_gather(ybuf, [av])
            za = plsc.load_gather(zbuf, [av])
            obase = srow * K
            rbase = srow * K * 3
            for t in range(4):
                kt = acc[2 * t]
                it = acc[2 * t + 1]
                idx_s[pl.ds(obase + t * 16, 16)] = it + _splat_i32(moff)
                mk = kt < CUT2
                msk_s[pl.ds(obase + t * 16, 16)] = mk.astype(jnp.int32)
                kc = jnp.maximum(kt, 1e-30)
                u = plsc.bitcast(kc, jnp.int32)
                y = plsc.bitcast(_splat_i32(0x5F3759DF) -
                                 lax.shift_right_logical(u, 1), jnp.float32)
                half = kc * (-0.5)
                for _ in range(3):
                    y = y * (half * y * y + 1.5)
                d = kt * y
                dist_s[pl.ds(obase + t * 16, 16)] = jnp.where(mk, d, 0.0)
                mf = jnp.where(mk, 1.0, 0.0)
                sb = _splat_i32(rbase + t * 48) + iota3
                xg = plsc.load_gather(xbuf, [it])
                plsc.store_scatter(rij_s, [sb], (xg - xa) * mf)
                yg = plsc.load_gather(ybuf, [it])
                plsc.store_scatter(rij_s, [sb + ones], (yg - ya) * mf)
                zg = plsc.load_gather(zbuf, [it])
                plsc.store_scatter(rij_s, [sb + ones + ones], (zg - za) * mf)
            return carry

        maxr0 = M * A - BLK
        cp0 = pltpu.make_async_copy(d2_hbm.at[pl.ds(r0, BLK)], inbuf0, sem0)
        cp0.start()

        def blk2_body(h, _):
            b0r = r0 + (2 * h) * BLK
            b1r = r0 + (2 * h + 1) * BLK
            b2r = jnp.minimum(r0 + (2 * h + 2) * BLK, maxr0)
            pltpu.make_async_copy(
                d2_hbm.at[pl.ds(b0r, BLK)], inbuf0, sem0).wait()
            pltpu.make_async_copy(
                d2_hbm.at[pl.ds(b1r, BLK)], inbuf1, sem1).start()
            def rb0(rb, c):
                row_body(rb, (c[0], inbuf0))
                return c
            lax.fori_loop(0, BLK, rb0, (b0r,))
            pltpu.make_async_copy(
                d2_hbm.at[pl.ds(b1r, BLK)], inbuf1, sem1).wait()
            pltpu.make_async_copy(
                d2_hbm.at[pl.ds(b2r, BLK)], inbuf0, sem0).start()
            def rb1(rb, c):
                row_body(rb, (c[0], inbuf1))
                return c
            lax.fori_loop(0, BLK, rb1, (b1r,))
            return 0
        lax.fori_loop(0, NBLK // 2, blk2_body, 0)
        pltpu.make_async_copy(d2_hbm.at[pl.ds(maxr0, BLK)], inbuf0, sem0).wait()

        pltpu.sync_copy(idx_s, idx_hbm.at[pl.ds(r0 * K, ROWS_PER_W * K)])
        pltpu.sync_copy(dist_s, dist_hbm.at[pl.ds(r0 * K, ROWS_PER_W * K)])
        pltpu.sync_copy(rij_s, rij_hbm.at[pl.ds(r0 * K * 3, ROWS_PER_W * K * 3)])
        pltpu.sync_copy(msk_s, msk_hbm.at[pl.ds(r0 * K, ROWS_PER_W * K)])

    return k(d2f, px, py, pz)


def kernel(atom_types, positions, n_atoms, cells, pbc, n_molecules):
    pos = positions.reshape(M, A, 3)
    sq = jnp.sum(pos * pos, axis=-1)
    d2 = _d2_tc(pos, jnp.swapaxes(pos, 1, 2), sq[..., None], sq[:, None, :])

    px = pos[:, :, 0].reshape(-1)
    py = pos[:, :, 1].reshape(-1)
    pz = pos[:, :, 2].reshape(-1)
    idx_j, dist, rij, msk = _sc_select(d2, px, py, pz)

    idx_i = jnp.repeat(jnp.arange(M * A, dtype=jnp.int32), K)
    return (idx_i,
            idx_j,
            rij.reshape(M, A, K, 3),
            dist.reshape(M, A, K),
            msk.reshape(M, A, K).astype(bool))


# speedup vs baseline: 1.1151x; 1.0019x over previous
import functools

import jax
import jax.numpy as jnp
from jax import lax
from jax.experimental import pallas as pl
from jax.experimental.pallas import tpu as pltpu
from jax.experimental.pallas import tpu_sc as plsc

M = 8
A = 1024
K = 64
CUT2 = 49.0
NBUCKET = 32
BUCKET_SCALE = NBUCKET / 160.0

NW = 32
ROWS_PER_W = (M * A) // NW
BLK = 8
NBLK = ROWS_PER_W // BLK


def _d2_tc(pos, posT, sqc, sqr):
    def body(p_ref, pt_ref, sc_ref, sr_ref, o_ref):
        dot = jnp.dot(p_ref[0], pt_ref[0], preferred_element_type=jnp.float32)
        d2 = (sc_ref[0] + sr_ref[0]) - 2.0 * dot
        d2 = jnp.maximum(d2, 0.0)
        r = lax.broadcasted_iota(jnp.int32, (A, A), 0)
        c = lax.broadcasted_iota(jnp.int32, (A, A), 1)
        o_ref[...] = d2 + jnp.where(r == c, 1e10, 0.0).astype(jnp.float32)

    return pl.pallas_call(
        body,
        grid=(M,),
        in_specs=[
            pl.BlockSpec((1, A, 3), lambda m: (m, 0, 0)),
            pl.BlockSpec((1, 3, A), lambda m: (m, 0, 0)),
            pl.BlockSpec((1, A, 1), lambda m: (m, 0, 0)),
            pl.BlockSpec((1, 1, A), lambda m: (m, 0, 0)),
        ],
        out_specs=pl.BlockSpec((A, A), lambda m: (m, 0)),
        out_shape=jax.ShapeDtypeStruct((M * A, A), jnp.float32),
    )(pos, posT, sqc, sqr)


def _splat_i32(x):
    return jnp.full((16,), 1, jnp.int32) * x


def _scalar(v16):
    return v16[0]


def _sc_select(d2f, px, py, pz):
    mesh = plsc.VectorSubcoreMesh(core_axis_name="c", subcore_axis_name="s",
                                  num_cores=2, num_subcores=16)

    @functools.partial(
        pl.kernel,
        out_type=[
            jax.ShapeDtypeStruct((M * A * K,), jnp.int32),
            jax.ShapeDtypeStruct((M * A * K,), jnp.float32),
            jax.ShapeDtypeStruct((M * A * K * 3,), jnp.float32),
            jax.ShapeDtypeStruct((M * A * K,), jnp.int32),
        ],
        mesh=mesh,
        scratch_types=[
            pltpu.VMEM((BLK, A), jnp.float32),
            pltpu.VMEM((BLK, A), jnp.float32),
            pltpu.VMEM((A,), jnp.float32),
            pltpu.VMEM((A,), jnp.float32),
            pltpu.VMEM((A,), jnp.float32),
            pltpu.VMEM((A + 16,), jnp.float32),
            pltpu.VMEM((A + 16,), jnp.int32),
            pltpu.VMEM((16 * NBUCKET,), jnp.int32),
            pltpu.VMEM((ROWS_PER_W * K,), jnp.int32),
            pltpu.VMEM((ROWS_PER_W * K,), jnp.float32),
            pltpu.VMEM((ROWS_PER_W * K * 3,), jnp.float32),
            pltpu.VMEM((ROWS_PER_W * K,), jnp.int32),
            pltpu.VMEM((K,), jnp.float32),
            pltpu.VMEM((K,), jnp.int32),
            pltpu.SemaphoreType.DMA,
            pltpu.SemaphoreType.DMA,
        ],
        compiler_params=pltpu.CompilerParams(needs_layout_passes=False,
                                             use_tc_tiling_on_sc=True),
    )
    def k(d2_hbm, px_hbm, py_hbm, pz_hbm,
          idx_hbm, dist_hbm, rij_hbm, msk_hbm,
          inbuf0, inbuf1, xbuf, ybuf, zbuf, keyb, idxb, cnt,
          idx_s, dist_s, rij_s, msk_s, fkb, fib, sem0, sem1):
        wid = lax.axis_index("s") * 2 + lax.axis_index("c")
        m = wid // 4
        r0 = wid * ROWS_PER_W
        moff = m * A

        iota = lax.iota(jnp.int32, 16)
        iota3 = iota * 3
        ones = jnp.full((16,), 1, jnp.int32)
        zeros16 = jnp.full((16,), 0, jnp.int32)
        inff = jnp.full((16,), 3.0e38, jnp.float32)

        pltpu.sync_copy(px_hbm.at[pl.ds(moff, A)], xbuf)
        pltpu.sync_copy(py_hbm.at[pl.ds(moff, A)], ybuf)
        pltpu.sync_copy(pz_hbm.at[pl.ds(moff, A)], zbuf)

        def row_body(rb, carry):
            blk_r0, inbuf = carry
            r = blk_r0 + rb
            a_loc = r - moff
            srow = r - r0

            def zero_body(i, _):
                cnt[pl.ds(i * 16, 16)] = zeros16
                return 0
            lax.fori_loop(0, 16 * NBUCKET // 16, zero_body, 0)

            laneoff = iota * NBUCKET
            def hist_body(c, _):
                v = inbuf[rb, pl.ds(c * 16, 16)]
                b = jnp.minimum(v * BUCKET_SCALE,
                                float(NBUCKET - 1)).astype(jnp.int32)
                addr = b + laneoff
                old = plsc.load_gather(cnt, [addr])
                plsc.store_scatter(cnt, [addr], old + ones)
                return 0
            lax.fori_loop(0, A // 16, hist_body, 0)

            def scan_body(i, carry2):
                tot, bfound = carry2
                def fold(l, s):
                    return s + cnt[pl.ds(l * NBUCKET + i * 16, 16)]
                ch = lax.fori_loop(0, 16, fold, zeros16)
                cum = plsc.cumsum(ch) + _splat_i32(tot)
                cross = cum >= K
                anyc = cum[15] >= K
                ffs = _scalar(plsc.all_reduce_ffs(cross))
                cand = i * 16 + ffs
                bnew = jnp.where(anyc & (bfound >= NBUCKET),
                                 cand, bfound)
                return cum[15], bnew
            _, bsel = lax.fori_loop(0, NBUCKET // 16, scan_body,
                                    (jnp.int32(0), jnp.int32(NBUCKET)))
            bsel_v = _splat_i32(bsel)

            def comp_body(c, off):
                v = inbuf[rb, pl.ds(c * 16, 16)]
                b = jnp.minimum(v * BUCKET_SCALE,
                                float(NBUCKET - 1)).astype(jnp.int32)
                mk = b <= bsel_v
                plsc.store_compressed(keyb.at[pl.ds(off, 16)], v, mask=mk)
                iv = iota + _splat_i32(c * 16)
                plsc.store_compressed(idxb.at[pl.ds(off, 16)], iv, mask=mk)
                pc = _scalar(plsc.all_reduce_population_count(mk))
                return off + pc
            csz = lax.fori_loop(0, A // 16, comp_body, jnp.int32(0))

            c0 = csz // 16
            rem = csz - c0 * 16
            remv = _splat_i32(rem)
            tk = keyb[pl.ds(c0 * 16, 16)]
            keyb[pl.ds(c0 * 16, 16)] = jnp.where(iota >= remv, inff, tk)
            ti = idxb[pl.ds(c0 * 16, 16)]
            idxb[pl.ds(c0 * 16, 16)] = jnp.where(iota >= remv, zeros16, ti)
            nch = (csz + 15) // 16

            def merge_body(c, acc):
                ak0, ai0, ak1, ai1, ak2, ai2, ak3, ai3 = acc
                ck, ci = plsc.sort_key_val(keyb[pl.ds(c * 16, 16)],
                                           idxb[pl.ds(c * 16, 16)])
                outs = []
                for (akj, aij) in ((ak0, ai0), (ak1, ai1),
                                   (ak2, ai2), (ak3, ai3)):
                    rk = lax.rev(ck, (0,))
                    ri = lax.rev(ci, (0,))
                    a_le = (akj < rk) | ((akj == rk) & (aij <= ri))
                    lo_k = jnp.where(a_le, akj, rk)
                    lo_i = jnp.where(a_le, aij, ri)
                    hi_k = jnp.where(a_le, rk, akj)
                    hi_i = jnp.where(a_le, ri, aij)
                    nk, ni = plsc.sort_key_val(lo_k, lo_i)
                    ck, ci = plsc.sort_key_val(hi_k, hi_i)
                    outs.append((nk, ni))
                return (outs[0][0], outs[0][1], outs[1][0], outs[1][1],
                        outs[2][0], outs[2][1], outs[3][0], outs[3][1])
            init = (inff, zeros16, inff, zeros16,
                    inff, zeros16, inff, zeros16)
            acc = lax.fori_loop(0, nch, merge_body, init)

            aks = (acc[0], acc[2], acc[4], acc[6])
            ais = (acc[1], acc[3], acc[5], acc[7])
            for t in range(4):
                fkb[pl.ds(t * 16, 16)] = aks[t]
                fib[pl.ds(t * 16, 16)] = ais[t]
            carry = jnp.int32(-1)
            key2 = []
            nties = jnp.int32(0)
            for t in range(4):
                pidx = jnp.maximum(iota + (16 * t - 1), 0)
                prev = plsc.load_gather(fkb, [pidx])
                newrun = aks[t] != prev
                if t == 0:
                    newrun = newrun | (iota < 1)
                nties = nties + _scalar(
                    plsc.all_reduce_population_count(~newrun))
                gp = iota + _splat_i32(16 * t)
                s = jnp.where(newrun, gp, -1)
                r0v = jnp.maximum(plsc.cummax(s), _splat_i32(carry))
                carry = r0v[15]
                key2.append(r0v * 2048 + ais[t])

            def fixup(ops):
                key2a, key2b, key2c, key2d = ops
                big = _splat_i32(1 << 30)
                b0, b1, b2, b3 = big, big, big, big
                p0 = p1 = p2 = p3 = zeros16
                for t, k2 in enumerate((key2a, key2b, key2c, key2d)):
                    gp = iota + _splat_i32(16 * t)
                    ck2, cp2 = plsc.sort_key_val(k2, gp)
                    for j in range(4):
                        bj = (b0, b1, b2, b3)[j]
                        pj = (p0, p1, p2, p3)[j]
                        rk2 = lax.rev(ck2, (0,))
                        rp2 = lax.rev(cp2, (0,))
                        a_le = bj <= rk2
                        lo_k = jnp.where(a_le, bj, rk2)
                        lo_p = jnp.where(a_le, pj, rp2)
                        hi_k = jnp.where(a_le, rk2, bj)
                        hi_p = jnp.where(a_le, rp2, pj)
                        nk2, np2 = plsc.sort_key_val(lo_k, lo_p)
                        ck2, cp2 = plsc.sort_key_val(hi_k, hi_p)
                        if j == 0:
                            b0, p0 = nk2, np2
                        elif j == 1:
                            b1, p1 = nk2, np2
                        elif j == 2:
                            b2, p2 = nk2, np2
                        else:
                            b3, p3 = nk2, np2
                    del ck2, cp2
                fin = []
                for pj in (p0, p1, p2, p3):
                    kf = plsc.load_gather(fkb, [pj])
                    vf = plsc.load_gather(fib, [pj])
                    fin.extend((kf, vf))
                return tuple(fin)

            acc = lax.cond(nties > 0, fixup, lambda ops: acc, tuple(key2))

            av = _splat_i32(a_loc)
            xa = plsc.load_gather(xbuf, [av])
            ya = plsc.load_gather(ybuf, [av])
            za = plsc.load_gather(zbuf, [av])
            obase = srow * K
            rbase = srow * K * 3
            for t in range(4):
                kt = acc[2 * t]
                it = acc[2 * t + 1]
                idx_s[pl.ds(obase + t * 16, 16)] = it + _splat_i32(moff)
                mk = kt < CUT2
                msk_s[pl.ds(obase + t * 16, 16)] = mk.astype(jnp.int32)
                kc = jnp.maximum(kt, 1e-30)
                u = plsc.bitcast(kc, jnp.int32)
                y = plsc.bitcast(_splat_i32(0x5F3759DF) -
                                 lax.shift_right_logical(u, 1), jnp.float32)
                half = kc * (-0.5)
                for _ in range(3):
                    y = y * (half * y * y + 1.5)
                d = kt * y
                dist_s[pl.ds(obase + t * 16, 16)] = jnp.where(mk, d, 0.0)
                mf = jnp.where(mk, 1.0, 0.0)
                sb = _splat_i32(rbase + t * 48) + iota3
                xg = plsc.load_gather(xbuf, [it])
                plsc.store_scatter(rij_s, [sb], (xg - xa) * mf)
                yg = plsc.load_gather(ybuf, [it])
                plsc.store_scatter(rij_s, [sb + ones], (yg - ya) * mf)
                zg = plsc.load_gather(zbuf, [it])
                plsc.store_scatter(rij_s, [sb + ones + ones], (zg - za) * mf)
            return carry

        maxr0 = M * A - BLK
        cp0 = pltpu.make_async_copy(d2_hbm.at[pl.ds(r0, BLK)], inbuf0, sem0)
        cp0.start()

        def blk2_body(h, _):
            b0r = r0 + (2 * h) * BLK
            b1r = r0 + (2 * h + 1) * BLK
            b2r = jnp.minimum(r0 + (2 * h + 2) * BLK, maxr0)
            pltpu.make_async_copy(
                d2_hbm.at[pl.ds(b0r, BLK)], inbuf0, sem0).wait()
            pltpu.make_async_copy(
                d2_hbm.at[pl.ds(b1r, BLK)], inbuf1, sem1).start()
            def rb0(rb, c):
                row_body(rb, (c[0], inbuf0))
                return c
            lax.fori_loop(0, BLK, rb0, (b0r,))
            pltpu.make_async_copy(
                d2_hbm.at[pl.ds(b1r, BLK)], inbuf1, sem1).wait()
            pltpu.make_async_copy(
                d2_hbm.at[pl.ds(b2r, BLK)], inbuf0, sem0).start()
            def rb1(rb, c):
                row_body(rb, (c[0], inbuf1))
                return c
            lax.fori_loop(0, BLK, rb1, (b1r,))
            return 0
        lax.fori_loop(0, NBLK // 2, blk2_body, 0)
        pltpu.make_async_copy(d2_hbm.at[pl.ds(maxr0, BLK)], inbuf0, sem0).wait()

        pltpu.sync_copy(idx_s, idx_hbm.at[pl.ds(r0 * K, ROWS_PER_W * K)])
        pltpu.sync_copy(dist_s, dist_hbm.at[pl.ds(r0 * K, ROWS_PER_W * K)])
        pltpu.sync_copy(rij_s, rij_hbm.at[pl.ds(r0 * K * 3, ROWS_PER_W * K * 3)])
        pltpu.sync_copy(msk_s, msk_hbm.at[pl.ds(r0 * K, ROWS_PER_W * K)])

    return k(d2f, px, py, pz)


def kernel(atom_types, positions, n_atoms, cells, pbc, n_molecules):
    pos = positions.reshape(M, A, 3)
    sq = jnp.sum(pos * pos, axis=-1)
    d2 = _d2_tc(pos, jnp.swapaxes(pos, 1, 2), sq[..., None], sq[:, None, :])

    px = pos[:, :, 0].reshape(-1)
    py = pos[:, :, 1].reshape(-1)
    pz = pos[:, :, 2].reshape(-1)
    idx_j, dist, rij, msk = _sc_select(d2, px, py, pz)

    idx_i = jnp.repeat(jnp.arange(M * A, dtype=jnp.int32), K)
    return (idx_i,
            idx_j,
            rij.reshape(M, A, K, 3),
            dist.reshape(M, A, K),
            msk.reshape(M, A, K).astype(bool))
